# baseline (speedup 1.0000x reference)
import functools

import jax
import jax.numpy as jnp
from jax import lax
from jax.experimental import pallas as pl
from jax.experimental.pallas import tpu as pltpu
from jax.experimental.pallas import tpu_sc as plsc

VOCAB = 1000
VOCAB_PAD = 1024
BATCH = 1024
SEQ = 50
SEQ_PAD = 56
NUM_WORKERS = 32
BPW = BATCH // NUM_WORKERS

_mesh = plsc.VectorSubcoreMesh(core_axis_name="c", subcore_axis_name="s")


@functools.partial(
    pl.kernel,
    mesh=_mesh,
    out_type=jax.ShapeDtypeStruct((BATCH, SEQ_PAD, VOCAB_PAD), jnp.float32),
    scratch_types=[
        pltpu.VMEM((BPW, SEQ_PAD), jnp.int32),
        pltpu.VMEM((SEQ_PAD, 8, 128), jnp.float32),
        pltpu.VMEM((SEQ_PAD, 8, 128), jnp.float32),
        pltpu.SemaphoreType.DMA,
        pltpu.SemaphoreType.DMA,
        pltpu.SemaphoreType.DMA,
        pltpu.SemaphoreType.DMA,
    ],
)
def _gather_rows(table_hbm, idx_hbm, out_hbm, idx_v, rows0, rows1,
                 sem0, sem1, psem0, psem1):
    wid = lax.axis_index("s") * 2 + lax.axis_index("c")
    base = wid * BPW

    pltpu.sync_copy(idx_hbm.at[pl.ds(base, BPW)], idx_v)

    def gather(b, rows_v, sem):
        return pltpu.async_copy(table_hbm.at[idx_v.at[b]], rows_v, sem)

    def gather_wait(b, rows_v, sem):
        pltpu.make_async_copy(table_hbm.at[idx_v.at[b]], rows_v, sem).wait()

    def put_start(b, rows_v, psem):
        for tc in range(8):
            pltpu.async_copy(rows_v.at[:, tc],
                             out_hbm.at[base + b].at[:, pl.ds(tc * 128, 128)],
                             psem)

    def put_wait(b, rows_v, psem):
        for tc in range(8):
            pltpu.make_async_copy(
                rows_v.at[:, tc],
                out_hbm.at[base + b].at[:, pl.ds(tc * 128, 128)],
                psem).wait()

    gather(0, rows0, sem0)
    gather(1, rows1, sem1)

    @pl.loop(0, BPW // 2)
    def _(j):
        b0 = j * 2
        gather_wait(b0, rows0, sem0)
        put_start(b0, rows0, psem0)
        gather_wait(b0 + 1, rows1, sem1)
        put_start(b0 + 1, rows1, psem1)
        put_wait(b0, rows0, psem0)

        @pl.when(j < BPW // 2 - 1)
        def _():
            gather(b0 + 2, rows0, sem0)

        put_wait(b0 + 1, rows1, psem1)

        @pl.when(j < BPW // 2 - 1)
        def _():
            gather(b0 + 3, rows1, sem1)


def kernel(x, embedding_weight):
    idx = jnp.pad(x.astype(jnp.int32), ((0, 0), (0, SEQ_PAD - SEQ)))
    table = jnp.pad(embedding_weight, ((0, 0), (0, VOCAB_PAD - VOCAB)))
    table = table.reshape(VOCAB, 8, 128)
    out = _gather_rows(table, idx)
    return out[:, :SEQ, :VOCAB]


# speedup vs baseline: 1.2622x; 1.2622x over previous
import functools

import jax
import jax.numpy as jnp
from jax import lax
from jax.experimental import pallas as pl
from jax.experimental.pallas import tpu as pltpu
from jax.experimental.pallas import tpu_sc as plsc

VOCAB = 1000
VOCAB_PAD = 1024
BATCH = 1024
SEQ = 50
NSPLIT = 4
BSPLIT = BATCH // NSPLIT
NUM_WORKERS = 32
BPW = BSPLIT // NUM_WORKERS

_mesh = plsc.VectorSubcoreMesh(core_axis_name="c", subcore_axis_name="s")


@functools.partial(
    pl.kernel,
    mesh=_mesh,
    out_type=jax.ShapeDtypeStruct((BSPLIT, SEQ, 8, 128), jnp.float32),
    scratch_types=[
        pltpu.VMEM((BPW, SEQ), jnp.int32),
        pltpu.VMEM((SEQ, 8, 128), jnp.float32),
        pltpu.VMEM((SEQ, 8, 128), jnp.float32),
        pltpu.SemaphoreType.DMA,
        pltpu.SemaphoreType.DMA,
    ],
)
def _gather_rows(table_hbm, idx_hbm, out_hbm, idx_v, rows0, rows1, sem0, sem1):
    wid = lax.axis_index("s") * 2 + lax.axis_index("c")
    base = wid * BPW

    pltpu.sync_copy(idx_hbm.at[pl.ds(base, BPW)], idx_v)

    def gather(b, rows_v, sem):
        return pltpu.async_copy(table_hbm.at[idx_v.at[b]], rows_v, sem)

    def gather_wait(b, rows_v, sem):
        pltpu.make_async_copy(table_hbm.at[idx_v.at[b]], rows_v, sem).wait()

    def put(b, rows_v):
        pltpu.sync_copy(rows_v, out_hbm.at[base + b])

    gather(0, rows0, sem0)

    @pl.loop(0, BPW // 2)
    def _(j):
        b0 = j * 2
        gather_wait(b0, rows0, sem0)
        gather(b0 + 1, rows1, sem1)
        put(b0, rows0)
        gather_wait(b0 + 1, rows1, sem1)

        @pl.when(j < BPW // 2 - 1)
        def _():
            gather(b0 + 2, rows0, sem0)

        put(b0 + 1, rows1)


def kernel(x, embedding_weight):
    idx = x.astype(jnp.int32)
    table = jnp.pad(embedding_weight, ((0, 0), (0, VOCAB_PAD - VOCAB)))
    table = table.reshape(VOCAB, 8, 128)
    parts = [
        _gather_rows(table, idx[k * BSPLIT:(k + 1) * BSPLIT])
        .reshape(BSPLIT, SEQ, VOCAB_PAD)[:, :, :VOCAB]
        for k in range(NSPLIT)
    ]
    return jnp.concatenate(parts, axis=0)


# speedup vs baseline: 1.5507x; 1.2286x over previous
import functools

import jax
import jax.numpy as jnp
from jax import lax
from jax.experimental import pallas as pl
from jax.experimental.pallas import tpu as pltpu
from jax.experimental.pallas import tpu_sc as plsc

VOCAB = 1000
VOCAB_PAD = 1024
BATCH = 1024
SEQ = 50
NUM_WORKERS = 32
BPW = BATCH // NUM_WORKERS

_mesh = plsc.VectorSubcoreMesh(core_axis_name="c", subcore_axis_name="s")


@functools.partial(
    pl.kernel,
    mesh=_mesh,
    out_type=jax.ShapeDtypeStruct((BATCH, SEQ, 8, 128), jnp.float32),
    scratch_types=[
        pltpu.VMEM((BPW, SEQ), jnp.int32),
        pltpu.VMEM((SEQ, 8, 128), jnp.float32),
        pltpu.VMEM((SEQ, 8, 128), jnp.float32),
        pltpu.SemaphoreType.DMA,
        pltpu.SemaphoreType.DMA,
    ],
)
def _gather_rows(table_hbm, idx_hbm, out_hbm, idx_v, rows0, rows1, sem0, sem1):
    wid = lax.axis_index("s") * 2 + lax.axis_index("c")
    base = wid * BPW

    pltpu.sync_copy(idx_hbm.at[pl.ds(base, BPW)], idx_v)

    def gather(b, rows_v, sem):
        return pltpu.async_copy(table_hbm.at[idx_v.at[b]], rows_v, sem)

    def gather_wait(b, rows_v, sem):
        pltpu.make_async_copy(table_hbm.at[idx_v.at[b]], rows_v, sem).wait()

    def put(b, rows_v):
        pltpu.sync_copy(rows_v, out_hbm.at[base + b])

    gather(0, rows0, sem0)

    @pl.loop(0, BPW // 2)
    def _(j):
        b0 = j * 2
        gather_wait(b0, rows0, sem0)
        gather(b0 + 1, rows1, sem1)
        put(b0, rows0)
        gather_wait(b0 + 1, rows1, sem1)

        @pl.when(j < BPW // 2 - 1)
        def _():
            gather(b0 + 2, rows0, sem0)

        put(b0 + 1, rows1)


def kernel(x, embedding_weight):
    idx = x.astype(jnp.int32)
    table = jnp.pad(embedding_weight, ((0, 0), (0, VOCAB_PAD - VOCAB)))
    table = table.reshape(VOCAB, 8, 128)
    out = _gather_rows(table, idx)
    return out.reshape(BATCH, SEQ, VOCAB_PAD)[:, :, :VOCAB]


# speedup vs baseline: 1.5520x; 1.0008x over previous
import functools

import jax
import jax.numpy as jnp
from jax import lax
from jax.experimental import pallas as pl
from jax.experimental.pallas import tpu as pltpu
from jax.experimental.pallas import tpu_sc as plsc

VOCAB = 1000
VOCAB_PAD = 1024
BATCH = 1024
SEQ = 50
NUM_WORKERS = 32
BPW = BATCH // NUM_WORKERS

_mesh = plsc.VectorSubcoreMesh(core_axis_name="c", subcore_axis_name="s")


@functools.partial(
    pl.kernel,
    mesh=_mesh,
    out_type=jax.ShapeDtypeStruct((BATCH, SEQ, 8, 128), jnp.float32),
    scratch_types=[
        pltpu.VMEM((BPW, SEQ), jnp.int32),
        pltpu.VMEM((SEQ, 8, 128), jnp.float32),
        pltpu.VMEM((SEQ, 8, 128), jnp.float32),
        pltpu.SemaphoreType.DMA,
        pltpu.SemaphoreType.DMA,
    ],
)
def _gather_rows(table_hbm, idx_hbm, out_hbm, idx_v, rows0, rows1, sem0, sem1):
    wid = lax.axis_index("s") * 2 + lax.axis_index("c")
    base = wid * BPW

    pltpu.sync_copy(idx_hbm.at[pl.ds(base, BPW)], idx_v)

    def gather(b, rows_v, sem):
        return pltpu.async_copy(table_hbm.at[idx_v.at[b]], rows_v, sem)

    def gather_wait(b, rows_v, sem):
        pltpu.make_async_copy(table_hbm.at[idx_v.at[b]], rows_v, sem).wait()

    def put(b, rows_v):
        pltpu.sync_copy(rows_v, out_hbm.at[base + b])

    gather(0, rows0, sem0)

    @pl.loop(0, BPW // 2)
    def _(j):
        b0 = j * 2
        gather_wait(b0, rows0, sem0)
        gather(b0 + 1, rows1, sem1)
        put(b0, rows0)
        gather_wait(b0 + 1, rows1, sem1)

        @pl.when(j < BPW // 2 - 1)
        def _():
            gather(b0 + 2, rows0, sem0)

        put(b0 + 1, rows1)


def kernel(x, embedding_weight):
    idx = x.astype(jnp.int32)
    table = jnp.pad(embedding_weight, ((0, 0), (0, VOCAB_PAD - VOCAB)))
    table = table.reshape(VOCAB, 8, 128)
    out = _gather_rows(table, idx)
    out = out.reshape(BATCH, SEQ, VOCAB_PAD)
    return lax.pad(out, jnp.float32(0),
                   ((0, 0, 0), (0, 0, 0), (0, VOCAB - VOCAB_PAD, 0)))
